# Initial kernel scaffold; baseline (speedup 1.0000x reference)
import jax
import jax.numpy as jnp
from jax.experimental import pallas as pl


def kernel(inputs, tables):
    raise NotImplementedError("write your pallas kernel here")


# speedup vs baseline: 2.4978x; 2.4978x over previous
import functools

import jax
import jax.numpy as jnp
from jax import lax
from jax.experimental import pallas as pl
from jax.experimental.pallas import tpu as pltpu
from jax.experimental.pallas import tpu_sc as plsc

_F = 26
_V = 104000
_D = 16
_B = 4096
_R = _B * _F * _F
_NC = 2
_NS = 16
_NW = _NC * _NS
_RPW = _R // _NW
_CH = 1352
_NCH = _RPW // _CH


def _gather_kernel(idx_hbm, tbl_hbm, out_hbm, idx_v, rows_v, gsem):
    wid = lax.axis_index("s") * _NC + lax.axis_index("c")
    base = wid * _RPW

    def chunk(c, carry):
        start = base + c * _CH
        pltpu.sync_copy(idx_hbm.at[pl.ds(start, _CH)], idx_v)
        pltpu.async_copy(tbl_hbm.at[idx_v], rows_v, gsem).wait()
        pltpu.sync_copy(rows_v, out_hbm.at[pl.ds(start, _CH)])
        return carry

    lax.fori_loop(0, _NCH, chunk, 0)


@jax.jit
def _run(idx, tbl):
    mesh = plsc.VectorSubcoreMesh(
        core_axis_name="c", subcore_axis_name="s", num_cores=_NC, num_subcores=_NS
    )
    f = functools.partial(
        pl.kernel,
        mesh=mesh,
        out_type=jax.ShapeDtypeStruct((_R, _D), jnp.float32),
        scratch_types=[
            pltpu.VMEM((_CH,), jnp.int32),
            pltpu.VMEM((_CH, _D), jnp.float32),
            pltpu.SemaphoreType.DMA,
        ],
        compiler_params=pltpu.CompilerParams(use_tc_tiling_on_sc=False),
    )(_gather_kernel)
    return f(idx, tbl)


def kernel(inputs, tables):
    field_off = jnp.arange(_F, dtype=jnp.int32) * 4000
    table_off = jnp.arange(_F, dtype=jnp.int32) * _V
    idx = (
        (inputs + field_off[None, :])[:, :, None] + table_off[None, None, :]
    ).reshape(_R)
    tbl = tables.reshape(_F * _V, _D)
    out = _run(idx, tbl)
    return out.reshape(_B, _F, _F, _D)


# speedup vs baseline: 10.4091x; 4.1673x over previous
import functools

import jax
import jax.numpy as jnp
from jax import lax
from jax.experimental import pallas as pl
from jax.experimental.pallas import tpu as pltpu
from jax.experimental.pallas import tpu_sc as plsc

_F = 26
_V = 104000
_D = 16
_B = 4096
_C = 4000
_NC = 2
_NS = 16
_NW = _NC * _NS
_NBLK = _F * _F
_BPW = -(-_NBLK // _NW)
_NVR = _B // 16


def _lookup_kernel(inT_hbm, tpl_hbm, out_hbm, idx_v, win_v, obuf_v):
    wid = lax.axis_index("s") * _NC + lax.axis_index("c")

    def blk_step(k, carry):
        blk = wid + _NW * k

        @pl.when(blk < _NBLK)
        def _():
            f = blk // _F
            t = blk % _F
            pltpu.sync_copy(inT_hbm.at[f], idx_v)
            pltpu.sync_copy(
                tpl_hbm.at[pl.ds(t * _D, _D), pl.ds(f * _C, _C)], win_v
            )

            def d_step(d, c2):
                dvec = jnp.full((16,), d, jnp.int32)

                def v_step(i, c3):
                    iv = idx_v[pl.ds(i * 16, 16)]
                    obuf_v[pl.ds(i * 16, 16)] = plsc.load_gather(
                        win_v, [dvec, iv]
                    )
                    return c3

                lax.fori_loop(0, _NVR, v_step, 0, unroll=8)
                pltpu.sync_copy(obuf_v, out_hbm.at[f, t, d])
                return c2

            lax.fori_loop(0, _D, d_step, 0)

        return carry

    lax.fori_loop(0, _BPW, blk_step, 0)


@jax.jit
def _run(inT, tpl):
    mesh = plsc.VectorSubcoreMesh(
        core_axis_name="c", subcore_axis_name="s", num_cores=_NC, num_subcores=_NS
    )
    f = functools.partial(
        pl.kernel,
        mesh=mesh,
        out_type=jax.ShapeDtypeStruct((_F, _F, _D, _B), jnp.float32),
        scratch_types=[
            pltpu.VMEM((_B,), jnp.int32),
            pltpu.VMEM((_D, _C), jnp.float32),
            pltpu.VMEM((_B,), jnp.float32),
        ],
        compiler_params=pltpu.CompilerParams(
            use_tc_tiling_on_sc=False, needs_layout_passes=False
        ),
    )(_lookup_kernel)
    return f(inT, tpl)


def kernel(inputs, tables):
    inT = jnp.transpose(inputs, (1, 0))
    tpl = jnp.transpose(tables, (0, 2, 1)).reshape(_F * _D, _V)
    out = _run(inT, tpl)
    return jnp.transpose(out, (3, 0, 1, 2))


# speedup vs baseline: 15.6493x; 1.5034x over previous
import functools

import jax
import jax.numpy as jnp
from jax import lax
from jax.experimental import pallas as pl
from jax.experimental.pallas import tpu as pltpu
from jax.experimental.pallas import tpu_sc as plsc

_F = 26
_V = 104000
_D = 16
_B = 4096
_C = 4000
_NC = 2
_NS = 16
_NW = _NC * _NS
_NBLK = _F * _F
_QD = 4
_NVR = _B // 16


def _task(wid, t):
    blk = wid + _NW * (t // _QD)
    q = t % _QD
    return blk // _F, blk % _F, q


def _win_src(tpl_hbm, f, tt, q):
    return tpl_hbm.at[pl.ds(tt * _D + q * _QD, _QD), pl.ds(f * _C, _C)]


def _lookup_kernel(inT_hbm, tpl_hbm, out_hbm, idx_v, win_v, obuf_v, ism, wsm, osm):
    wid = lax.axis_index("s") * _NC + lax.axis_index("c")
    nvalid = jnp.where(wid < _NBLK - 21 * _NW, 88, 84)

    def issue_in(b, t):
        f, tt, q = _task(wid, t)
        pltpu.async_copy(inT_hbm.at[f], idx_v.at[b], ism.at[b])
        pltpu.async_copy(_win_src(tpl_hbm, f, tt, q), win_v.at[b], wsm.at[b])

    for b in range(2):
        issue_in(b, b)

    def body(k, carry):
        for b in range(2):
            t = 2 * k + b

            @pl.when(t < nvalid)
            def _():
                f, tt, q = _task(wid, t)
                pltpu.make_async_copy(inT_hbm.at[f], idx_v.at[b], ism.at[b]).wait()
                pltpu.make_async_copy(
                    _win_src(tpl_hbm, f, tt, q), win_v.at[b], wsm.at[b]
                ).wait()
                for d4 in range(_QD):
                    s = _QD * b + d4
                    d = q * _QD + d4
                    dst = out_hbm.at[f, tt, d]

                    @pl.when(k >= 1)
                    def _drain():
                        pltpu.make_async_copy(obuf_v.at[s], dst, osm.at[s]).wait()

                    dvec = jnp.full((16,), d4, jnp.int32)

                    def v_step(i, c3):
                        iv = idx_v.at[b][pl.ds(i * 16, 16)]
                        obuf_v.at[s][pl.ds(i * 16, 16)] = plsc.load_gather(
                            win_v.at[b], [dvec, iv]
                        )
                        return c3

                    lax.fori_loop(0, _NVR, v_step, 0, unroll=8)
                    pltpu.async_copy(obuf_v.at[s], dst, osm.at[s])

                @pl.when(t + 2 < nvalid)
                def _prefetch():
                    issue_in(b, t + 2)

        return carry

    lax.fori_loop(0, 44, body, 0)

    for s in range(2 * _QD):
        pltpu.make_async_copy(obuf_v.at[s], out_hbm.at[0, 0, 0], osm.at[s]).wait()


@jax.jit
def _run(inT, tpl):
    mesh = plsc.VectorSubcoreMesh(
        core_axis_name="c", subcore_axis_name="s", num_cores=_NC, num_subcores=_NS
    )
    f = functools.partial(
        pl.kernel,
        mesh=mesh,
        out_type=jax.ShapeDtypeStruct((_F, _F, _D, _B), jnp.float32),
        scratch_types=[
            pltpu.VMEM((2, _B), jnp.int32),
            pltpu.VMEM((2, _QD, _C), jnp.float32),
            pltpu.VMEM((2 * _QD, _B), jnp.float32),
            pltpu.SemaphoreType.DMA((2,)),
            pltpu.SemaphoreType.DMA((2,)),
            pltpu.SemaphoreType.DMA((2 * _QD,)),
        ],
        compiler_params=pltpu.CompilerParams(
            use_tc_tiling_on_sc=False, needs_layout_passes=False
        ),
    )(_lookup_kernel)
    return f(inT, tpl)


def kernel(inputs, tables):
    inT = jnp.transpose(inputs, (1, 0))
    tpl = jnp.transpose(tables, (0, 2, 1)).reshape(_F * _D, _V)
    out = _run(inT, tpl)
    return jnp.transpose(out, (3, 0, 1, 2))


# speedup vs baseline: 25.7177x; 1.6434x over previous
import functools

import jax
import jax.numpy as jnp
from jax import lax
from jax.experimental import pallas as pl
from jax.experimental.pallas import tpu as pltpu
from jax.experimental.pallas import tpu_sc as plsc

_F = 26
_V = 104000
_D = 16
_B = 4096
_C = 4000
_NC = 2
_NS = 16
_NW = _NC * _NS
_NBLK = _F * _F
_QD = 4
_NVR = _B // 16


def _task(wid, t):
    blk = wid + _NW * (t // _QD)
    q = t % _QD
    return blk // _F, blk % _F, q


def _win_src(tpl_hbm, f, tt, q):
    return tpl_hbm.at[pl.ds(tt * _D + q * _QD, _QD), pl.ds(f * _C, _C)]


def _lookup_kernel(inT_hbm, tpl_hbm, out_hbm, idx_v, win_v, obuf_v, ism, wsm, osm):
    wid = lax.axis_index("s") * _NC + lax.axis_index("c")
    nvalid = jnp.where(wid < _NBLK - 21 * _NW, 88, 84)

    def issue_in(b, t):
        f, tt, q = _task(wid, t)
        pltpu.async_copy(inT_hbm.at[f], idx_v.at[b], ism.at[b])
        pltpu.async_copy(_win_src(tpl_hbm, f, tt, q), win_v.at[b], wsm.at[b])

    for b in range(2):
        issue_in(b, b)

    def body(k, carry):
        for b in range(2):
            t = 2 * k + b

            @pl.when(t < nvalid)
            def _():
                f, tt, q = _task(wid, t)
                pltpu.make_async_copy(inT_hbm.at[f], idx_v.at[b], ism.at[b]).wait()
                pltpu.make_async_copy(
                    _win_src(tpl_hbm, f, tt, q), win_v.at[b], wsm.at[b]
                ).wait()
                @pl.when(k >= 1)
                def _drain():
                    for d4 in range(_QD):
                        s = _QD * b + d4
                        pltpu.make_async_copy(
                            obuf_v.at[s], out_hbm.at[f, tt, q * _QD + d4], osm.at[s]
                        ).wait()

                dvecs = [jnp.full((16,), d4, jnp.int32) for d4 in range(_QD)]

                @plsc.parallel_loop(0, _NVR, 1, unroll=4)
                def v_step(i):
                    iv = idx_v.at[b][pl.ds(i * 16, 16)]
                    for d4 in range(_QD):
                        obuf_v.at[_QD * b + d4][pl.ds(i * 16, 16)] = (
                            plsc.load_gather(win_v.at[b], [dvecs[d4], iv])
                        )

                for d4 in range(_QD):
                    pltpu.async_copy(
                        obuf_v.at[_QD * b + d4],
                        out_hbm.at[f, tt, q * _QD + d4],
                        osm.at[_QD * b + d4],
                    )

                @pl.when(t + 2 < nvalid)
                def _prefetch():
                    issue_in(b, t + 2)

        return carry

    lax.fori_loop(0, 44, body, 0)

    for s in range(2 * _QD):
        pltpu.make_async_copy(obuf_v.at[s], out_hbm.at[0, 0, 0], osm.at[s]).wait()


@jax.jit
def _run(inT, tpl):
    mesh = plsc.VectorSubcoreMesh(
        core_axis_name="c", subcore_axis_name="s", num_cores=_NC, num_subcores=_NS
    )
    f = functools.partial(
        pl.kernel,
        mesh=mesh,
        out_type=jax.ShapeDtypeStruct((_F, _F, _D, _B), jnp.float32),
        scratch_types=[
            pltpu.VMEM((2, _B), jnp.int32),
            pltpu.VMEM((2, _QD, _C), jnp.float32),
            pltpu.VMEM((2 * _QD, _B), jnp.float32),
            pltpu.SemaphoreType.DMA((2,)),
            pltpu.SemaphoreType.DMA((2,)),
            pltpu.SemaphoreType.DMA((2 * _QD,)),
        ],
        compiler_params=pltpu.CompilerParams(
            use_tc_tiling_on_sc=False, needs_layout_passes=False
        ),
    )(_lookup_kernel)
    return f(inT, tpl)


def kernel(inputs, tables):
    inT = jnp.transpose(inputs, (1, 0))
    tpl = jnp.transpose(tables, (0, 2, 1)).reshape(_F * _D, _V)
    out = _run(inT, tpl)
    return jnp.transpose(out, (3, 0, 1, 2))


# speedup vs baseline: 40.0227x; 1.5562x over previous
import functools

import jax
import jax.numpy as jnp
from jax import lax
from jax.experimental import pallas as pl
from jax.experimental.pallas import tpu as pltpu
from jax.experimental.pallas import tpu_sc as plsc

_F = 26
_V = 104000
_D = 16
_B = 4096
_C = 4000
_NC = 2
_NS = 16
_NW = _NC * _NS
_NBLK = _F * _F
_HD = 8
_BB = _B // 128


def _win_src(tpl_hbm, f, tt, q):
    return tpl_hbm.at[pl.ds(tt * _D + q * _HD, _HD), pl.ds(f * _C, _C)]


def _lookup_kernel(inT_hbm, tpl_hbm, out_hbm, idx_v, win_v, obuf_v, ism, wsm, osm):
    wid = lax.axis_index("s") * _NC + lax.axis_index("c")
    nvalid = jnp.where(wid < _NBLK - 21 * _NW, 44, 42)

    def issue_in(b, t):
        blk = wid + _NW * (t // 2)
        f, tt = blk // _F, blk % _F
        pltpu.async_copy(inT_hbm.at[f], idx_v.at[b], ism.at[b])
        pltpu.async_copy(_win_src(tpl_hbm, f, tt, b), win_v.at[b], wsm.at[b])

    for b in range(2):
        issue_in(b, b)

    def body(k, carry):
        for b in range(2):
            t = 2 * k + b
            blk = wid + _NW * k
            f, tt = blk // _F, blk % _F

            @pl.when(t < nvalid)
            def _():
                pltpu.make_async_copy(inT_hbm.at[f], idx_v.at[b], ism.at[b]).wait()
                pltpu.make_async_copy(
                    _win_src(tpl_hbm, f, tt, b), win_v.at[b], wsm.at[b]
                ).wait()

                def drain(d8):
                    pltpu.make_async_copy(
                        obuf_v.at[d8],
                        out_hbm.at[f, tt, b, :, d8, :],
                        osm.at[d8],
                    ).wait()

                if b == 1:
                    for d8 in range(_HD):
                        drain(d8)
                else:

                    @pl.when(k >= 1)
                    def _dr():
                        for d8 in range(_HD):
                            drain(d8)

                dvecs = [jnp.full((16,), d8, jnp.int32) for d8 in range(_HD)]

                @plsc.parallel_loop(0, _BB, 1)
                def v_step(i):
                    for j in range(8):
                        iv = idx_v.at[b][pl.ds(i * 128 + j * 16, 16)]
                        for d8 in range(_HD):
                            obuf_v.at[d8, :, :][i, pl.ds(j * 16, 16)] = (
                                plsc.load_gather(win_v.at[b], [dvecs[d8], iv])
                            )

                for d8 in range(_HD):
                    pltpu.async_copy(
                        obuf_v.at[d8],
                        out_hbm.at[f, tt, b, :, d8, :],
                        osm.at[d8],
                    )

                @pl.when(t + 2 < nvalid)
                def _prefetch():
                    issue_in(b, t + 2)

        return carry

    lax.fori_loop(0, 22, body, 0)

    for s in range(_HD):
        pltpu.make_async_copy(
            obuf_v.at[s], out_hbm.at[0, 0, 0, :, s, :], osm.at[s]
        ).wait()


@jax.jit
def _run(inT, tpl):
    mesh = plsc.VectorSubcoreMesh(
        core_axis_name="c", subcore_axis_name="s", num_cores=_NC, num_subcores=_NS
    )
    f = functools.partial(
        pl.kernel,
        mesh=mesh,
        out_type=jax.ShapeDtypeStruct((_F, _F, 2, _BB, _HD, 128), jnp.float32),
        scratch_types=[
            pltpu.VMEM((2, _B), jnp.int32),
            pltpu.VMEM((2, _HD, _C), jnp.float32),
            pltpu.VMEM((_HD, _BB, 128), jnp.float32),
            pltpu.SemaphoreType.DMA((2,)),
            pltpu.SemaphoreType.DMA((2,)),
            pltpu.SemaphoreType.DMA((_HD,)),
        ],
        compiler_params=pltpu.CompilerParams(
            use_tc_tiling_on_sc=False, needs_layout_passes=False
        ),
    )(_lookup_kernel)
    return f(inT, tpl)


def kernel(inputs, tables):
    inT = jnp.transpose(inputs, (1, 0))
    tpl = jnp.transpose(tables, (0, 2, 1)).reshape(_F * _D, _V)
    out6 = _run(inT, tpl)
    out = jnp.transpose(out6, (3, 5, 0, 1, 2, 4))
    return out.reshape(_B, _F, _F, _D)
